# baseline (speedup 1.0000x reference)
import functools

import jax
import jax.numpy as jnp
import numpy as np
from jax.experimental import pallas as pl

EMBED_DIM = 1408
NUM_POS_FEATS = EMBED_DIM // 2
INPUT_IMAGE_SIZE = (16, 224, 224)
_N = 50


def _body(t_ref, x_ref, y_ref, lab_ref, g_ref, e0_ref, e1_ref, nap_ref, out_ref):
    T, H, W = INPUT_IMAGE_SIZE
    two_pi = 2.0 * np.pi
    def _r(v):
        return v.astype(jnp.bfloat16).astype(jnp.float32)

    a = _r(t_ref[...] / T * 2.0 - 1.0)
    b = _r(x_ref[...] / W * 2.0 - 1.0)
    c = _r(y_ref[...] / H * 2.0 - 1.0)
    g0 = _r(g_ref[:, 0:1, :])
    g1 = _r(g_ref[:, 1:2, :])
    g2 = _r(g_ref[:, 2:3, :])
    ph = (a * g0 + b * g1 + c * g2) * two_pi
    is1 = lab_ref[...] == 1
    add_s = jnp.where(is1, e1_ref[:, :, 0:NUM_POS_FEATS], e0_ref[:, :, 0:NUM_POS_FEATS])
    add_c = jnp.where(is1, e1_ref[:, :, NUM_POS_FEATS:], e0_ref[:, :, NUM_POS_FEATS:])
    out_ref[:, 0:_N, 0:NUM_POS_FEATS] = jnp.sin(ph) + add_s
    out_ref[:, 0:_N, NUM_POS_FEATS:EMBED_DIM] = jnp.cos(ph) + add_c
    bB = out_ref.shape[0]
    out_ref[:, _N:_N + 1, :] = jnp.broadcast_to(nap_ref[...], (bB, 1, EMBED_DIM))


@functools.partial(jax.jit, static_argnames=("block_b", "interpret"))
def _run(coords, labels, gaussian_matrix, point_embed_0, point_embed_1,
         not_a_point_embed, block_b=8, interpret=False):
    Bsz, N, _ = coords.shape
    t = coords[:, :, 0:1]
    x = coords[:, :, 1:2]
    y = coords[:, :, 2:3]
    lab = labels[:, :, None]
    g = gaussian_matrix[None]
    e0 = point_embed_0[None]
    e1 = point_embed_1[None]
    nap = not_a_point_embed[None]
    grid = (Bsz // block_b,)
    in_specs = [
        pl.BlockSpec((block_b, N, 1), lambda i: (i, 0, 0)),
        pl.BlockSpec((block_b, N, 1), lambda i: (i, 0, 0)),
        pl.BlockSpec((block_b, N, 1), lambda i: (i, 0, 0)),
        pl.BlockSpec((block_b, N, 1), lambda i: (i, 0, 0)),
        pl.BlockSpec((1, 3, NUM_POS_FEATS), lambda i: (0, 0, 0)),
        pl.BlockSpec((1, 1, EMBED_DIM), lambda i: (0, 0, 0)),
        pl.BlockSpec((1, 1, EMBED_DIM), lambda i: (0, 0, 0)),
        pl.BlockSpec((1, 1, EMBED_DIM), lambda i: (0, 0, 0)),
    ]
    out_spec = pl.BlockSpec((block_b, N + 1, EMBED_DIM), lambda i: (i, 0, 0))
    out_shape = jax.ShapeDtypeStruct((Bsz, N + 1, EMBED_DIM), jnp.float32)
    return pl.pallas_call(
        _body,
        grid=grid,
        in_specs=in_specs,
        out_specs=out_spec,
        out_shape=out_shape,
        interpret=interpret,
    )(t, x, y, lab, g, e0, e1, nap)


def kernel(coords, labels, gaussian_matrix, point_embed_0, point_embed_1,
           not_a_point_embed):
    return _run(coords, labels, gaussian_matrix, point_embed_0,
                point_embed_1, not_a_point_embed)


# speedup vs baseline: 1.8725x; 1.8725x over previous
import functools

import jax
import jax.numpy as jnp
import numpy as np
from jax.experimental import pallas as pl

EMBED_DIM = 1408
NUM_POS_FEATS = EMBED_DIM // 2
INPUT_IMAGE_SIZE = (16, 224, 224)
_N = 50


_PS = (3.141592640157309, -5.167710086059465, 2.5500775571606873,
       -0.5982913934957941, 0.07765765729608731)
_PC = (0.999999999780127, -4.934802137102469, 4.0587091596414036,
       -1.3352119964803657, 0.2349372493949808, -0.02439616627050223)
def _body(t_ref, x_ref, y_ref, lab_ref, g_ref, e0_ref, e1_ref, nap_ref, out_ref):
    T, H, W = INPUT_IMAGE_SIZE
    f32, i32 = jnp.float32, jnp.int32
    def _r(v):
        return v.astype(jnp.bfloat16).astype(f32)

    a = _r(t_ref[...] / T * 2.0 - 1.0)
    b = _r(x_ref[...] / W * 2.0 - 1.0)
    c = _r(y_ref[...] / H * 2.0 - 1.0)
    g0 = _r(g_ref[:, 0:1, :]) * 2.0
    g1 = _r(g_ref[:, 1:2, :]) * 2.0
    g2 = _r(g_ref[:, 2:3, :]) * 2.0
    d2 = a * g0 + b * g1 + c * g2
    k = jnp.round(d2)
    ki = k.astype(i32)
    u = d2 - k
    z = u * u
    ps = _PS[0] + z * (_PS[1] + z * (_PS[2] + z * (_PS[3] + z * _PS[4])))
    ps = u * ps
    pc = _PC[0] + z * (_PC[1] + z * (_PC[2] + z * (_PC[3] + z * (_PC[4] + z * _PC[5]))))
    sgn = jax.lax.shift_left(jax.lax.bitwise_and(ki, 1), 31)
    s = jax.lax.bitcast_convert_type(
        jax.lax.bitwise_xor(jax.lax.bitcast_convert_type(ps, i32), sgn), f32)
    co = jax.lax.bitcast_convert_type(
        jax.lax.bitwise_xor(jax.lax.bitcast_convert_type(pc, i32), sgn), f32)
    is1 = lab_ref[...] == 1
    add_s = jnp.where(is1, e1_ref[:, :, 0:NUM_POS_FEATS], e0_ref[:, :, 0:NUM_POS_FEATS])
    add_c = jnp.where(is1, e1_ref[:, :, NUM_POS_FEATS:], e0_ref[:, :, NUM_POS_FEATS:])
    out_ref[:, 0:_N, 0:NUM_POS_FEATS] = s + add_s
    out_ref[:, 0:_N, NUM_POS_FEATS:EMBED_DIM] = co + add_c
    bB = out_ref.shape[0]
    out_ref[:, _N:_N + 1, :] = jnp.broadcast_to(nap_ref[...], (bB, 1, EMBED_DIM))


@functools.partial(jax.jit, static_argnames=("block_b", "interpret"))
def _run(coords, labels, gaussian_matrix, point_embed_0, point_embed_1,
         not_a_point_embed, block_b=8, interpret=False):
    Bsz, N, _ = coords.shape
    t = coords[:, :, 0:1]
    x = coords[:, :, 1:2]
    y = coords[:, :, 2:3]
    lab = labels[:, :, None]
    g = gaussian_matrix[None]
    e0 = point_embed_0[None]
    e1 = point_embed_1[None]
    nap = not_a_point_embed[None]
    grid = (Bsz // block_b,)
    in_specs = [
        pl.BlockSpec((block_b, N, 1), lambda i: (i, 0, 0)),
        pl.BlockSpec((block_b, N, 1), lambda i: (i, 0, 0)),
        pl.BlockSpec((block_b, N, 1), lambda i: (i, 0, 0)),
        pl.BlockSpec((block_b, N, 1), lambda i: (i, 0, 0)),
        pl.BlockSpec((1, 3, NUM_POS_FEATS), lambda i: (0, 0, 0)),
        pl.BlockSpec((1, 1, EMBED_DIM), lambda i: (0, 0, 0)),
        pl.BlockSpec((1, 1, EMBED_DIM), lambda i: (0, 0, 0)),
        pl.BlockSpec((1, 1, EMBED_DIM), lambda i: (0, 0, 0)),
    ]
    out_spec = pl.BlockSpec((block_b, N + 1, EMBED_DIM), lambda i: (i, 0, 0))
    out_shape = jax.ShapeDtypeStruct((Bsz, N + 1, EMBED_DIM), jnp.float32)
    return pl.pallas_call(
        _body,
        grid=grid,
        in_specs=in_specs,
        out_specs=out_spec,
        out_shape=out_shape,
        interpret=interpret,
    )(t, x, y, lab, g, e0, e1, nap)


def kernel(coords, labels, gaussian_matrix, point_embed_0, point_embed_1,
           not_a_point_embed):
    return _run(coords, labels, gaussian_matrix, point_embed_0,
                point_embed_1, not_a_point_embed)
